# Initial kernel scaffold; baseline (speedup 1.0000x reference)
import jax
import jax.numpy as jnp
from jax.experimental import pallas as pl


def kernel(x, table):
    raise NotImplementedError("write your pallas kernel here")


# speedup vs baseline: 2.1424x; 2.1424x over previous
import jax
import jax.numpy as jnp
from jax.experimental import pallas as pl


_EB = 512
_LB = 512


def _body(x_ref, t_ref, o_ref):
    t = t_ref[...]
    o_ref[...] = x_ref[...] + t.T[None, :, :]


def kernel(x, table):
    b, e, l = x.shape
    grid = (e // _EB, l // _LB)
    return pl.pallas_call(
        _body,
        grid=grid,
        in_specs=[
            pl.BlockSpec((b, _EB, _LB), lambda ei, li: (0, ei, li)),
            pl.BlockSpec((_LB, _EB), lambda ei, li: (li, ei)),
        ],
        out_specs=pl.BlockSpec((b, _EB, _LB), lambda ei, li: (0, ei, li)),
        out_shape=jax.ShapeDtypeStruct(x.shape, x.dtype),
    )(x, table)


# speedup vs baseline: 2.2469x; 1.0488x over previous
import jax
import jax.numpy as jnp
from jax.experimental import pallas as pl


_EB = 128
_LB = 4096


def _body(x_ref, t_ref, o_ref):
    t = t_ref[...]
    o_ref[...] = x_ref[...] + t.T[None, :, :]


def kernel(x, table):
    b, e, l = x.shape
    grid = (e // _EB, l // _LB)
    return pl.pallas_call(
        _body,
        grid=grid,
        in_specs=[
            pl.BlockSpec((b, _EB, _LB), lambda ei, li: (0, ei, li)),
            pl.BlockSpec((_LB, _EB), lambda ei, li: (li, ei)),
        ],
        out_specs=pl.BlockSpec((b, _EB, _LB), lambda ei, li: (0, ei, li)),
        out_shape=jax.ShapeDtypeStruct(x.shape, x.dtype),
    )(x, table)
